# baseline (speedup 1.0000x reference)
import functools

import jax
import jax.numpy as jnp
import numpy as np
from jax.experimental import pallas as pl

N_NODES = 50000
NE = 800000
D = 100


def _mm_kernel(x_ref, w_ref, o_ref):
    o_ref[...] = jax.nn.leaky_relu(
        jnp.dot(x_ref[...], w_ref[...].T, preferred_element_type=jnp.float32),
        negative_slope=0.01)


def _feats_matmul(x, Wx):
    m, k = x.shape
    n = Wx.shape[0]
    bm = 2000
    return pl.pallas_call(
        _mm_kernel,
        grid=(m // bm,),
        in_specs=[
            pl.BlockSpec((bm, k), lambda i: (i, 0)),
            pl.BlockSpec((n, k), lambda i: (0, 0)),
        ],
        out_specs=pl.BlockSpec((bm, n), lambda i: (i, 0)),
        out_shape=jax.ShapeDtypeStruct((m, n), jnp.float32),
    )(x, Wx)


def kernel(x, edge_index_all, rel_emb, r_index, line_graph_index, line_graph_val, Wx, Wr, Ww1):
    feats = _feats_matmul(x, Wx)

    k = edge_index_all[0].astype(jnp.uint32) * np.uint32(N_NODES) \
        + edge_index_all[1].astype(jnp.uint32)
    sk = jnp.sort(k)
    first = jnp.concatenate([jnp.array([True]), sk[1:] != sk[:-1]])
    pos = jnp.cumsum(first) - 1
    n_u = pos[-1] + 1
    ukey = jnp.zeros((NE,), jnp.uint32).at[pos].set(sk)
    iota = jnp.arange(NE)
    valid = iota < n_u
    adj1 = jnp.where(valid, (ukey // N_NODES).astype(jnp.int32), 0)
    adj2 = jnp.where(valid, (ukey % N_NODES).astype(jnp.int32), 0)

    rel = rel_emb @ Wr.T
    r0, r1 = r_index[0], r_index[1]
    counts = jnp.zeros((NE,), jnp.float32).at[r0].add(1.0)
    r_val = 1.0 / (counts[r0] + 1e-16)
    rels_sum = jnp.zeros((NE, D), jnp.float32).at[r0].add(
        r_val[:, None] * rel[r1 % rel.shape[0]])
    nrm = jnp.linalg.norm(rels_sum, axis=1, keepdims=True)
    u = rels_sum / jnp.maximum(nrm, 1e-12)

    w1, w2, w3 = Ww1[0, :D], Ww1[0, D:2 * D], Ww1[0, 2 * D:3 * D]
    s1 = feats @ w1
    s2 = feats @ w2
    uw2 = u @ w2
    uw3 = u @ w3
    nrows = feats[adj2]
    dot = jnp.sum(nrows * u, axis=1)
    logit = s1[adj1] + s2[adj2] - 2.0 * dot * uw2 + uw3
    logit = jnp.where(valid, logit, -jnp.inf)

    m = jax.ops.segment_max(logit, adj1, num_segments=N_NODES)
    m = jnp.where(jnp.isfinite(m), m, 0.0)
    e = jnp.exp(logit - m[adj1])
    dsum = jax.ops.segment_sum(e, adj1, num_segments=N_NODES)
    att = e / (dsum[adj1] + 1e-16)

    coef2 = -2.0 * att * dot
    new = jnp.zeros((N_NODES, D), jnp.float32).at[adj1].add(
        att[:, None] * nrows + coef2[:, None] * u)
    return jnp.concatenate(
        [feats, jax.nn.leaky_relu(new, negative_slope=0.01)], axis=1)


# speedup vs baseline: 4.0260x; 4.0260x over previous
import functools

import jax
import jax.numpy as jnp
import numpy as np
from jax import lax
from jax.experimental import pallas as pl
from jax.experimental.pallas import tpu as pltpu
from jax.experimental.pallas import tpu_sc as plsc

N_NODES = 50000
NE = 800000
D = 100
DP = 128
NV = DP // 16
C = 128
EC = 128
OB = 64
NW = 32
PAD = 256


def _mm_kernel(act, x_ref, w_ref, o_ref):
    r = jnp.dot(x_ref[...], w_ref[...].T, preferred_element_type=jnp.float32)
    if act:
        r = jax.nn.leaky_relu(r, negative_slope=0.01)
    o_ref[:, :D] = r
    o_ref[:, D:] = jnp.zeros_like(o_ref[:, D:])


def _matmul_pad(x, W, act, bm):
    m, k = x.shape
    return pl.pallas_call(
        functools.partial(_mm_kernel, act),
        grid=(m // bm,),
        in_specs=[
            pl.BlockSpec((bm, k), lambda i: (i, 0)),
            pl.BlockSpec((D, k), lambda i: (0, 0)),
        ],
        out_specs=pl.BlockSpec((bm, DP), lambda i: (i, 0)),
        out_shape=jax.ShapeDtypeStruct((m, DP), jnp.float32),
    )(x, W)


def _rsqrt16(x):
    i = lax.bitcast_convert_type(x, jnp.int32)
    y = lax.bitcast_convert_type(jnp.int32(0x5F3759DF) - (i >> 1), jnp.float32)
    for _ in range(3):
        y = y * (1.5 - 0.5 * x * y * y)
    return y


def _splat(s):
    return jnp.full((16,), s, jnp.float32)


def _lsum(v):
    ii = lax.iota(jnp.int32, 16)
    for k in (1, 2, 4, 8):
        v = v + v.at[ii ^ k].get(mode="promise_in_bounds")
    return v


def _edge_body(a1s_hbm, cnts_hbm, adj2_hbm, r1s_hbm, parms_hbm, feats_hbm,
               rel_hbm, w23_hbm, out_hbm,
               parms_v, w23_v, a1_v, cnt_v, a2_v, r1_v, frows_v, rrows_v,
               obuf_v, sem):
    cid = lax.axis_index("c")
    sid = lax.axis_index("s")
    wid = sid * 2 + cid

    pltpu.sync_copy(parms_hbm, parms_v)
    pltpu.sync_copy(w23_hbm, w23_v)

    def _sca(j):
        return parms_v[j, pl.ds(wid, 16)][0]

    e_lo = _sca(0)
    e_hi = _sca(1)
    p0 = _sca(2)
    n_lo = _sca(3)
    n_hi = _sca(4)

    w2 = [w23_v[pl.ds(v * 16, 16)] for v in range(NV)]
    w3 = [w23_v[pl.ds(DP + v * 16, 16)] for v in range(NV)]

    zero = _splat(0.0)
    MNEG = -1e30
    mneg = _splat(MNEG)
    o7 = tuple(zero for _ in range(NV))

    def finalize_to(cur, stop, st):
        @pl.loop(cur, stop, init_carry=st)
        def fin(nd, stf):
            obi, obase, m, s, O = stf
            inv = 1.0 / (s + 1e-16)
            base = obi * DP
            for v in range(NV):
                r = O[v] * inv
                obuf_v[pl.ds(base + v * 16, 16)] = jnp.where(
                    r >= 0.0, r, 0.01 * r)
            obi = obi + 1
            full = (obi == OB).astype(jnp.int32)

            @pl.loop(0, full, init_carry=0)
            def _fl(_, d):
                pltpu.async_copy(
                    obuf_v,
                    out_hbm.at[pl.ds(pl.multiple_of(obase * DP, 8),
                                     OB * DP)],
                    sem).wait()
                return d

            obi = obi * (1 - full)
            obase = obase + OB * full
            return (obi, obase, mneg, zero, o7)

        return fin

    jlo = e_lo >> 7
    jhi = (e_hi + (C - 1)) >> 7

    st0 = (p0, p0 - EC, n_lo, 0, n_lo, mneg, zero, o7)

    @pl.loop(jlo, jhi, init_carry=st0)
    def chunk(j, st):
        p, pb, cur, obi, obase, m, s, O = st
        eb = pl.multiple_of(j * C, C)
        pltpu.async_copy(a1s_hbm.at[pl.ds(eb, C + 16)], a1_v, sem).wait()
        pltpu.async_copy(cnts_hbm.at[pl.ds(eb, C + 16)], cnt_v, sem).wait()
        pltpu.async_copy(adj2_hbm.at[pl.ds(eb, C)], a2_v, sem).wait()
        pltpu.async_copy(feats_hbm.at[a2_v], frows_v, sem).wait()
        elo_j = jnp.maximum(e_lo, eb)
        ehi_j = jnp.minimum(e_hi, eb + C)

        @pl.loop(elo_j, ehi_j, init_carry=(p, pb, cur, obi, obase, m, s, O))
        def edges(e, st2):
            p, pb, cur, obi, obase, m, s, O = st2
            i = e - eb
            a1 = a1_v[pl.ds(i, 16)][0]
            cnt = cnt_v[pl.ds(i, 16)][0]

            obi, obase, m, s, O = finalize_to(
                cur, a1, (obi, obase, m, s, O))
            cur = a1

            @pl.loop(0, cnt, init_carry=(p, pb, o7))
            def ent(k, st3):
                p, pb, acc = st3
                need = (p - pb >= EC).astype(jnp.int32)

                @pl.loop(0, need, init_carry=pb)
                def refill(_, pbx):
                    npb = pl.multiple_of(p & ~7, 8)
                    pltpu.async_copy(r1s_hbm.at[pl.ds(npb, EC)], r1_v,
                                     sem).wait()
                    pltpu.async_copy(rel_hbm.at[r1_v], rrows_v, sem).wait()
                    return npb

                pb = refill
                bp = p - pb
                acc = tuple(acc[v] + rrows_v[bp, pl.ds(v * 16, 16)]
                            for v in range(NV))
                return p + 1, pb, acc

            p, pb, acc = ent

            n2 = zero
            for v in range(NV):
                n2 = n2 + acc[v] * acc[v]
            n2 = _lsum(n2)
            inv = jnp.where(n2 <= 1e-30, 0.0, _rsqrt16(n2))
            u = [acc[v] * inv for v in range(NV)]

            row = [frows_v[i, pl.ds(v * 16, 16)] for v in range(NV)]
            t1 = zero
            t2 = zero
            t3 = zero
            t4 = zero
            for v in range(NV):
                t1 = t1 + row[v] * u[v]
                t2 = t2 + row[v] * w2[v]
                t3 = t3 + u[v] * w2[v]
                t4 = t4 + u[v] * w3[v]
            d1 = _lsum(t1)
            d2 = _lsum(t2)
            uw2 = _lsum(t3)
            uw3 = _lsum(t4)
            l = d2 - 2.0 * d1 * uw2 + uw3

            mn = jnp.maximum(m, l)
            sc = jnp.exp(m - mn)
            w = jnp.exp(l - mn)
            s = s * sc + w
            d1x2 = 2.0 * d1
            O = tuple(O[v] * sc + w * (row[v] - d1x2 * u[v])
                      for v in range(NV))
            return p, pb, cur, obi, obase, mn, s, O

        return edges

    p, pb, cur, obi, obase, m, s, O = chunk
    obi, obase, m, s, O = finalize_to(cur, n_hi, (obi, obase, m, s, O))

    @pl.loop(0, obi, init_carry=0, step=8)
    def drain(jr, d):
        pltpu.async_copy(
            obuf_v.at[pl.ds(pl.multiple_of(jr * DP, 8), 8 * DP)],
            out_hbm.at[pl.ds(pl.multiple_of((obase + jr) * DP, 8), 8 * DP)],
            sem).wait()
        return d


def _edge_pass(a1s, cnts, adj2s, r1s, parms, feats_pad, rel_pad, w23):
    mesh = plsc.VectorSubcoreMesh(core_axis_name="c", subcore_axis_name="s")
    f = pl.kernel(
        _edge_body,
        mesh=mesh,
        out_type=jax.ShapeDtypeStruct((N_NODES * DP,), jnp.float32),
        scratch_types=[
            pltpu.VMEM((5, NW + 16), jnp.int32),
            pltpu.VMEM((2 * DP,), jnp.float32),
            pltpu.VMEM((C + 16,), jnp.int32),
            pltpu.VMEM((C + 16,), jnp.int32),
            pltpu.VMEM((C,), jnp.int32),
            pltpu.VMEM((EC,), jnp.int32),
            pltpu.VMEM((C, DP), jnp.float32),
            pltpu.VMEM((EC, DP), jnp.float32),
            pltpu.VMEM((OB * DP,), jnp.float32),
            pltpu.SemaphoreType.DMA,
        ],
    )
    return f(a1s, cnts, adj2s, r1s, parms, feats_pad, rel_pad, w23)


def kernel(x, edge_index_all, rel_emb, r_index, line_graph_index,
           line_graph_val, Wx, Wr, Ww1):
    feats_pad = _matmul_pad(x, Wx, True, 2000)
    rel_pad = _matmul_pad(rel_emb, Wr, False, 2000)
    feats = feats_pad[:, :D]

    key = edge_index_all[0].astype(jnp.uint32) * np.uint32(N_NODES) \
        + edge_index_all[1].astype(jnp.uint32)
    sk = jnp.sort(key)
    first = jnp.concatenate([jnp.array([True]), sk[1:] != sk[:-1]])
    pos = jnp.cumsum(first) - 1
    n_u = (pos[-1] + 1).astype(jnp.int32)
    ukey = jnp.zeros((NE,), jnp.uint32).at[pos].set(sk)
    iota = jnp.arange(NE)
    valid = iota < n_u
    adj1s = (ukey // N_NODES).astype(jnp.int32)
    adj2s = (ukey % N_NODES).astype(jnp.int32)
    adj1pad = jnp.where(valid, adj1s, N_NODES)

    r0, r1 = r_index[0], r_index[1]
    cnts = jnp.zeros((NE,), jnp.int32).at[r0].add(1)
    rkey = jnp.sort(r0.astype(jnp.int32) * 2048 + (r1 % 2000).astype(jnp.int32))
    r0s = rkey >> 11
    r1s = rkey & 2047

    tt = jnp.arange(NW + 1)
    tgt = tt * n_u // NW
    cand = jnp.where(tgt >= n_u, N_NODES,
                     adj1pad[jnp.clip(tgt, 0, NE - 1)])
    nsplit = jnp.where(tt == 0, 0, (cand // 8) * 8).astype(jnp.int32)
    esplit = jnp.searchsorted(adj1pad, nsplit, side="left").astype(jnp.int32)
    rstart = jnp.searchsorted(r0s, esplit, side="left").astype(jnp.int32)
    parms = jnp.stack([esplit[:NW], esplit[1:], rstart[:NW],
                       nsplit[:NW], nsplit[1:]]).astype(jnp.int32)
    parms = jnp.pad(parms, ((0, 0), (0, 16)))

    zpad = jnp.zeros((PAD,), jnp.int32)
    a1sp = jnp.concatenate([adj1pad, zpad])
    cntsp = jnp.concatenate([cnts, zpad])
    adj2sp = jnp.concatenate([adj2s, zpad])
    r1sp = jnp.concatenate([r1s, zpad])
    w23 = jnp.concatenate([
        jnp.pad(Ww1[0, D:2 * D], (0, DP - D)),
        jnp.pad(Ww1[0, 2 * D:3 * D], (0, DP - D))])

    new_flat = _edge_pass(a1sp, cntsp, adj2sp, r1sp, parms, feats_pad,
                          rel_pad, w23)
    new_pad = new_flat.reshape(N_NODES, DP)
    return jnp.concatenate([feats, new_pad[:, :D]], axis=1)


# speedup vs baseline: 5.8279x; 1.4476x over previous
import functools

import jax
import jax.numpy as jnp
import numpy as np
from jax import lax
from jax.experimental import pallas as pl
from jax.experimental.pallas import tpu as pltpu
from jax.experimental.pallas import tpu_sc as plsc

N_NODES = 50000
NE = 800000
D = 100
DP = 128
NV = DP // 16
C = 128
EC = 128
OB = 64
NW = 32
PAD = 256


def _mm_kernel(act, x_ref, w_ref, o_ref):
    r = jnp.dot(x_ref[...], w_ref[...].T, preferred_element_type=jnp.float32)
    if act:
        r = jax.nn.leaky_relu(r, negative_slope=0.01)
    o_ref[:, :D] = r
    o_ref[:, D:] = jnp.zeros_like(o_ref[:, D:])


def _matmul_pad(x, W, act, bm):
    m, k = x.shape
    return pl.pallas_call(
        functools.partial(_mm_kernel, act),
        grid=(m // bm,),
        in_specs=[
            pl.BlockSpec((bm, k), lambda i: (i, 0)),
            pl.BlockSpec((D, k), lambda i: (0, 0)),
        ],
        out_specs=pl.BlockSpec((bm, DP), lambda i: (i, 0)),
        out_shape=jax.ShapeDtypeStruct((m, DP), jnp.float32),
    )(x, W)


def _rsqrt16(x):
    i = lax.bitcast_convert_type(x, jnp.int32)
    y = lax.bitcast_convert_type(jnp.int32(0x5F3759DF) - (i >> 1), jnp.float32)
    for _ in range(3):
        y = y * (1.5 - 0.5 * x * y * y)
    return y


def _splat(s):
    return jnp.full((16,), s, jnp.float32)


def _lsum(v):
    ii = lax.iota(jnp.int32, 16)
    for k in (1, 2, 4, 8):
        v = v + v.at[ii ^ k].get(mode="promise_in_bounds")
    return v


def _edge_body(a1s_hbm, cnts_hbm, adj2_hbm, r1s_hbm, parms_hbm, feats_hbm,
               rel_hbm, w23_hbm, out_hbm,
               parms_v, w23_v, a1_v, cnt_v, a2_v, r1_v, frows_v, rrows_v,
               obuf_v, sem, sem2):
    cid = lax.axis_index("c")
    sid = lax.axis_index("s")
    wid = sid * 2 + cid

    pltpu.sync_copy(parms_hbm, parms_v)
    pltpu.sync_copy(w23_hbm, w23_v)

    def _sca(j):
        return parms_v[j, pl.ds(wid, 16)][0]

    e_lo = _sca(0)
    e_hi = _sca(1)
    p0 = _sca(2)
    n_lo = _sca(3)
    n_hi = _sca(4)

    w2 = [w23_v[pl.ds(v * 16, 16)] for v in range(NV)]
    w3 = [w23_v[pl.ds(DP + v * 16, 16)] for v in range(NV)]

    zero = _splat(0.0)
    MNEG = -1e30
    mneg = _splat(MNEG)
    o7 = tuple(zero for _ in range(NV))

    def finalize_to(cur, stop, st):
        @pl.loop(cur, stop, init_carry=st)
        def fin(nd, stf):
            obi, obase, m, s, O = stf
            inv = 1.0 / (s + 1e-16)
            base = obi * DP
            for v in range(NV):
                r = O[v] * inv
                obuf_v[pl.ds(base + v * 16, 16)] = jnp.where(
                    r >= 0.0, r, 0.01 * r)
            obi = obi + 1
            full = (obi == OB).astype(jnp.int32)

            @pl.loop(0, full, init_carry=0)
            def _fl(_, d):
                pltpu.async_copy(
                    obuf_v,
                    out_hbm.at[pl.ds(pl.multiple_of(obase * DP, 8),
                                     OB * DP)],
                    sem).wait()
                return d

            obi = obi * (1 - full)
            obase = obase + OB * full
            return (obi, obase, mneg, zero, o7)

        return fin

    jlo = e_lo >> 7
    jhi = (e_hi + (C - 1)) >> 7

    st0 = (p0, p0 - EC, n_lo, 0, n_lo, mneg, zero, o7)

    @pl.loop(jlo, jhi, init_carry=st0)
    def chunk(j, st):
        p, pb, cur, obi, obase, m, s, O = st
        eb = pl.multiple_of(j * C, C)
        h3 = pltpu.async_copy(adj2_hbm.at[pl.ds(eb, C)], a2_v, sem2)
        h1 = pltpu.async_copy(a1s_hbm.at[pl.ds(eb, C + 16)], a1_v, sem)
        h2 = pltpu.async_copy(cnts_hbm.at[pl.ds(eb, C + 16)], cnt_v, sem)
        h3.wait()
        h4 = pltpu.async_copy(feats_hbm.at[a2_v], frows_v, sem2)
        h1.wait()
        h2.wait()
        h4.wait()
        elo_j = jnp.maximum(e_lo, eb)
        ehi_j = jnp.minimum(e_hi, eb + C)

        @pl.loop(elo_j, ehi_j, init_carry=(p, pb, cur, obi, obase, m, s, O))
        def edges(e, st2):
            p, pb, cur, obi, obase, m, s, O = st2
            i = e - eb
            a1 = a1_v[pl.ds(i, 16)][0]
            cnt = cnt_v[pl.ds(i, 16)][0]

            obi, obase, m, s, O = finalize_to(
                cur, a1, (obi, obase, m, s, O))
            cur = a1

            @pl.loop(0, cnt, init_carry=(p, pb, o7))
            def ent(k, st3):
                p, pb, acc = st3
                need = (p - pb >= EC).astype(jnp.int32)

                @pl.loop(0, need, init_carry=pb)
                def refill(_, pbx):
                    npb = pl.multiple_of(p & ~7, 8)
                    pltpu.async_copy(r1s_hbm.at[pl.ds(npb, EC)], r1_v,
                                     sem).wait()
                    pltpu.async_copy(rel_hbm.at[r1_v], rrows_v, sem).wait()
                    return npb

                pb = refill
                bp = p - pb
                acc = tuple(acc[v] + rrows_v[bp, pl.ds(v * 16, 16)]
                            for v in range(NV))
                return p + 1, pb, acc

            p, pb, acc = ent

            n2 = zero
            for v in range(NV):
                n2 = n2 + acc[v] * acc[v]
            n2 = _lsum(n2)
            inv = jnp.where(n2 <= 1e-30, 0.0, _rsqrt16(n2))
            u = [acc[v] * inv for v in range(NV)]

            row = [frows_v[i, pl.ds(v * 16, 16)] for v in range(NV)]
            t1 = zero
            t2 = zero
            t3 = zero
            t4 = zero
            for v in range(NV):
                t1 = t1 + row[v] * u[v]
                t2 = t2 + row[v] * w2[v]
                t3 = t3 + u[v] * w2[v]
                t4 = t4 + u[v] * w3[v]
            d1 = _lsum(t1)
            d2 = _lsum(t2)
            uw2 = _lsum(t3)
            uw3 = _lsum(t4)
            l = d2 - 2.0 * d1 * uw2 + uw3

            mn = jnp.maximum(m, l)
            sc = jnp.exp(m - mn)
            w = jnp.exp(l - mn)
            s = s * sc + w
            d1x2 = 2.0 * d1
            O = tuple(O[v] * sc + w * (row[v] - d1x2 * u[v])
                      for v in range(NV))
            return p, pb, cur, obi, obase, mn, s, O

        return edges

    p, pb, cur, obi, obase, m, s, O = chunk
    obi, obase, m, s, O = finalize_to(cur, n_hi, (obi, obase, m, s, O))

    @pl.loop(0, obi, init_carry=0, step=8)
    def drain(jr, d):
        pltpu.async_copy(
            obuf_v.at[pl.ds(pl.multiple_of(jr * DP, 8), 8 * DP)],
            out_hbm.at[pl.ds(pl.multiple_of((obase + jr) * DP, 8), 8 * DP)],
            sem).wait()
        return d


def _edge_pass(a1s, cnts, adj2s, r1s, parms, feats_pad, rel_pad, w23):
    mesh = plsc.VectorSubcoreMesh(core_axis_name="c", subcore_axis_name="s")
    f = pl.kernel(
        _edge_body,
        mesh=mesh,
        out_type=jax.ShapeDtypeStruct((N_NODES * DP,), jnp.float32),
        scratch_types=[
            pltpu.VMEM((5, NW + 16), jnp.int32),
            pltpu.VMEM((2 * DP,), jnp.float32),
            pltpu.VMEM((C + 16,), jnp.int32),
            pltpu.VMEM((C + 16,), jnp.int32),
            pltpu.VMEM((C,), jnp.int32),
            pltpu.VMEM((EC,), jnp.int32),
            pltpu.VMEM((C, DP), jnp.float32),
            pltpu.VMEM((EC, DP), jnp.float32),
            pltpu.VMEM((OB * DP,), jnp.float32),
            pltpu.SemaphoreType.DMA,
            pltpu.SemaphoreType.DMA,
        ],
    )
    return f(a1s, cnts, adj2s, r1s, parms, feats_pad, rel_pad, w23)


def kernel(x, edge_index_all, rel_emb, r_index, line_graph_index,
           line_graph_val, Wx, Wr, Ww1):
    feats_pad = _matmul_pad(x, Wx, True, 2000)
    rel_pad = _matmul_pad(rel_emb, Wr, False, 2000)
    feats = feats_pad[:, :D]

    key = edge_index_all[0].astype(jnp.uint32) * np.uint32(N_NODES) \
        + edge_index_all[1].astype(jnp.uint32)
    sk = jnp.sort(key)
    first = jnp.concatenate([jnp.array([True]), sk[1:] != sk[:-1]])
    pos = jnp.cumsum(first) - 1
    n_u = (pos[-1] + 1).astype(jnp.int32)
    a1sort = (sk // N_NODES).astype(jnp.int32)
    a2sort = (sk % N_NODES).astype(jnp.int32)
    zi = jnp.zeros((NE,), jnp.int32)
    adj1s = zi.at[pos].add(jnp.where(first, a1sort, 0))
    adj2s = zi.at[pos].add(jnp.where(first, a2sort, 0))
    iota = jnp.arange(NE)
    valid = iota < n_u
    adj1pad = jnp.where(valid, adj1s, N_NODES)

    r0, r1 = r_index[0], r_index[1]
    cnts = jnp.zeros((NE,), jnp.int32).at[r0].add(1)
    rkey = jnp.sort(r0.astype(jnp.int32) * 2048 + (r1 % 2000).astype(jnp.int32))
    r0s = rkey >> 11
    r1s = rkey & 2047

    tt = jnp.arange(NW + 1)
    tgt = tt * n_u // NW
    cand = jnp.where(tgt >= n_u, N_NODES,
                     adj1pad[jnp.clip(tgt, 0, NE - 1)])
    nsplit = jnp.where(tt == 0, 0, (cand // 8) * 8).astype(jnp.int32)
    esplit = jnp.searchsorted(adj1pad, nsplit, side="left").astype(jnp.int32)
    rstart = jnp.searchsorted(r0s, esplit, side="left").astype(jnp.int32)
    parms = jnp.stack([esplit[:NW], esplit[1:], rstart[:NW],
                       nsplit[:NW], nsplit[1:]]).astype(jnp.int32)
    parms = jnp.pad(parms, ((0, 0), (0, 16)))

    zpad = jnp.zeros((PAD,), jnp.int32)
    a1sp = jnp.concatenate([adj1pad, zpad])
    cntsp = jnp.concatenate([cnts, zpad])
    adj2sp = jnp.concatenate([adj2s, zpad])
    r1sp = jnp.concatenate([r1s, zpad])
    w23 = jnp.concatenate([
        jnp.pad(Ww1[0, D:2 * D], (0, DP - D)),
        jnp.pad(Ww1[0, 2 * D:3 * D], (0, DP - D))])

    new_flat = _edge_pass(a1sp, cntsp, adj2sp, r1sp, parms, feats_pad,
                          rel_pad, w23)
    new_pad = new_flat.reshape(N_NODES, DP)
    return jnp.concatenate([feats, new_pad[:, :D]], axis=1)


# speedup vs baseline: 6.0234x; 1.0335x over previous
import functools

import jax
import jax.numpy as jnp
import numpy as np
from jax import lax
from jax.experimental import pallas as pl
from jax.experimental.pallas import tpu as pltpu
from jax.experimental.pallas import tpu_sc as plsc

N_NODES = 50000
NE = 800000
D = 100
DP = 128
NV = DP // 16
C = 128
EC = 128
OB = 64
NW = 32
PAD = 256


def _mm_kernel(act, x_ref, w_ref, o_ref):
    r = jnp.dot(x_ref[...], w_ref[...].T, preferred_element_type=jnp.float32)
    if act:
        r = jax.nn.leaky_relu(r, negative_slope=0.01)
    o_ref[:, :D] = r
    o_ref[:, D:] = jnp.zeros_like(o_ref[:, D:])


def _matmul_pad(x, W, act, bm):
    m, k = x.shape
    return pl.pallas_call(
        functools.partial(_mm_kernel, act),
        grid=(m // bm,),
        in_specs=[
            pl.BlockSpec((bm, k), lambda i: (i, 0)),
            pl.BlockSpec((D, k), lambda i: (0, 0)),
        ],
        out_specs=pl.BlockSpec((bm, DP), lambda i: (i, 0)),
        out_shape=jax.ShapeDtypeStruct((m, DP), jnp.float32),
    )(x, W)


def _rsqrt16(x):
    i = lax.bitcast_convert_type(x, jnp.int32)
    y = lax.bitcast_convert_type(jnp.int32(0x5F3759DF) - (i >> 1), jnp.float32)
    for _ in range(3):
        y = y * (1.5 - 0.5 * x * y * y)
    return y


def _splat(s):
    return jnp.full((16,), s, jnp.float32)


def _lsum(v):
    ii = lax.iota(jnp.int32, 16)
    for k in (1, 2, 4, 8):
        v = v + v.at[ii ^ k].get(mode="promise_in_bounds")
    return v


def _edge_body(a1s_hbm, cnts_hbm, adj2_hbm, r1s_hbm, parms_hbm, feats_hbm,
               rel_hbm, w23_hbm, out_hbm,
               parms_v, w23_v, a1_v, cnt_v, a2_v, r1_v, frows_v, rrows_v,
               obuf_v, sem, sem2, sem3):
    cid = lax.axis_index("c")
    sid = lax.axis_index("s")
    wid = sid * 2 + cid

    pltpu.sync_copy(parms_hbm, parms_v)
    pltpu.sync_copy(w23_hbm, w23_v)

    def _sca(j):
        return parms_v[j, pl.ds(wid, 16)][0]

    e_lo = _sca(0)
    e_hi = _sca(1)
    p0 = _sca(2)
    n_lo = _sca(3)
    n_hi = _sca(4)

    w2 = [w23_v[pl.ds(v * 16, 16)] for v in range(NV)]
    w3 = [w23_v[pl.ds(DP + v * 16, 16)] for v in range(NV)]

    zero = _splat(0.0)
    MNEG = -1e30
    mneg = _splat(MNEG)
    o7 = tuple(zero for _ in range(NV))
    CB = C + 16

    def finalize_to(cur, stop, st):
        @pl.loop(cur, stop, init_carry=st)
        def fin(nd, stf):
            obi, obase, m, s, O = stf
            inv = 1.0 / (s + 1e-16)
            base = obi * DP
            for v in range(NV):
                r = O[v] * inv
                obuf_v[pl.ds(base + v * 16, 16)] = jnp.where(
                    r >= 0.0, r, 0.01 * r)
            obi = obi + 1
            full = (obi == OB).astype(jnp.int32)

            @pl.loop(0, full, init_carry=0)
            def _fl(_, d):
                pltpu.async_copy(
                    obuf_v,
                    out_hbm.at[pl.ds(pl.multiple_of(obase * DP, 8),
                                     OB * DP)],
                    sem3).wait()
                return d

            obi = obi * (1 - full)
            obase = obase + OB * full
            return (obi, obase, mneg, zero, o7)

        return fin

    def issue(jx):
        par = jx & 1
        ebx = pl.multiple_of(jx * C, C)
        pltpu.async_copy(a1s_hbm.at[pl.ds(ebx, CB)],
                         a1_v.at[pl.ds(par * CB, CB)], sem.at[par])
        pltpu.async_copy(cnts_hbm.at[pl.ds(ebx, CB)],
                         cnt_v.at[pl.ds(par * CB, CB)], sem.at[par])
        pltpu.async_copy(adj2_hbm.at[pl.ds(ebx, C)],
                         a2_v.at[pl.ds(par * C, C)], sem2.at[par]).wait()
        pltpu.async_copy(feats_hbm.at[a2_v.at[pl.ds(par * C, C)]],
                         frows_v.at[pl.ds(par * C, C), :], sem2.at[par])

    def wait_chunk(jx):
        par = jx & 1
        ebx = pl.multiple_of(jx * C, C)
        pltpu.make_async_copy(a1s_hbm.at[pl.ds(ebx, CB)],
                              a1_v.at[pl.ds(par * CB, CB)],
                              sem.at[par]).wait()
        pltpu.make_async_copy(cnts_hbm.at[pl.ds(ebx, CB)],
                              cnt_v.at[pl.ds(par * CB, CB)],
                              sem.at[par]).wait()
        pltpu.make_async_copy(feats_hbm.at[a2_v.at[pl.ds(par * C, C)]],
                              frows_v.at[pl.ds(par * C, C), :],
                              sem2.at[par]).wait()

    jlo = e_lo >> 7
    jhi = (e_hi + (C - 1)) >> 7

    @pl.loop(jlo, jnp.minimum(jlo + 1, jhi), init_carry=0)
    def _prol(jx, d):
        issue(jx)
        return d

    st0 = (p0, p0 - EC, n_lo, 0, n_lo, mneg, zero, o7)

    @pl.loop(jlo, jhi, init_carry=st0)
    def chunk(j, st):
        p, pb, cur, obi, obase, m, s, O = st
        par = j & 1
        eb = pl.multiple_of(j * C, C)

        @pl.loop(j + 1, jnp.minimum(j + 2, jhi), init_carry=0)
        def _pref(jx, d):
            issue(jx)
            return d

        wait_chunk(j)
        abase = par * CB + 0
        fbase = par * C
        elo_j = jnp.maximum(e_lo, eb)
        ehi_j = jnp.minimum(e_hi, eb + C)

        @pl.loop(elo_j, ehi_j, init_carry=(p, pb, cur, obi, obase, m, s, O))
        def edges(e, st2):
            p, pb, cur, obi, obase, m, s, O = st2
            i = e - eb
            a1 = a1_v[pl.ds(abase + i, 16)][0]
            cnt = cnt_v[pl.ds(abase + i, 16)][0]

            obi, obase, m, s, O = finalize_to(
                cur, a1, (obi, obase, m, s, O))
            cur = a1

            @pl.loop(0, cnt, init_carry=(p, pb, o7))
            def ent(k, st3):
                p, pb, acc = st3
                need = (p - pb >= EC).astype(jnp.int32)

                @pl.loop(0, need, init_carry=pb)
                def refill(_, pbx):
                    npb = pl.multiple_of(p & ~7, 8)
                    pltpu.async_copy(r1s_hbm.at[pl.ds(npb, EC)], r1_v,
                                     sem3).wait()
                    pltpu.async_copy(rel_hbm.at[r1_v], rrows_v,
                                     sem3).wait()
                    return npb

                pb = refill
                bp = p - pb
                acc = tuple(acc[v] + rrows_v[bp, pl.ds(v * 16, 16)]
                            for v in range(NV))
                return p + 1, pb, acc

            p, pb, acc = ent

            n2 = zero
            for v in range(NV):
                n2 = n2 + acc[v] * acc[v]
            n2 = _lsum(n2)
            inv = jnp.where(n2 <= 1e-30, 0.0, _rsqrt16(n2))
            u = [acc[v] * inv for v in range(NV)]

            row = [frows_v[fbase + i, pl.ds(v * 16, 16)]
                   for v in range(NV)]
            t1 = zero
            t2 = zero
            t3 = zero
            t4 = zero
            for v in range(NV):
                t1 = t1 + row[v] * u[v]
                t2 = t2 + row[v] * w2[v]
                t3 = t3 + u[v] * w2[v]
                t4 = t4 + u[v] * w3[v]
            d1 = _lsum(t1)
            d2 = _lsum(t2)
            uw2 = _lsum(t3)
            uw3 = _lsum(t4)
            l = d2 - 2.0 * d1 * uw2 + uw3

            mn = jnp.maximum(m, l)
            sc = jnp.exp(m - mn)
            w = jnp.exp(l - mn)
            s = s * sc + w
            d1x2 = 2.0 * d1
            O = tuple(O[v] * sc + w * (row[v] - d1x2 * u[v])
                      for v in range(NV))
            return p, pb, cur, obi, obase, mn, s, O

        return edges

    p, pb, cur, obi, obase, m, s, O = chunk
    obi, obase, m, s, O = finalize_to(cur, n_hi, (obi, obase, m, s, O))

    @pl.loop(0, obi, init_carry=0, step=8)
    def drain(jr, d):
        pltpu.async_copy(
            obuf_v.at[pl.ds(pl.multiple_of(jr * DP, 8), 8 * DP)],
            out_hbm.at[pl.ds(pl.multiple_of((obase + jr) * DP, 8), 8 * DP)],
            sem3).wait()
        return d


def _edge_pass(a1s, cnts, adj2s, r1s, parms, feats_pad, rel_pad, w23):
    mesh = plsc.VectorSubcoreMesh(core_axis_name="c", subcore_axis_name="s")
    f = pl.kernel(
        _edge_body,
        mesh=mesh,
        out_type=jax.ShapeDtypeStruct((N_NODES * DP,), jnp.float32),
        scratch_types=[
            pltpu.VMEM((5, NW + 16), jnp.int32),
            pltpu.VMEM((2 * DP,), jnp.float32),
            pltpu.VMEM((2 * (C + 16),), jnp.int32),
            pltpu.VMEM((2 * (C + 16),), jnp.int32),
            pltpu.VMEM((2 * C,), jnp.int32),
            pltpu.VMEM((EC,), jnp.int32),
            pltpu.VMEM((2 * C, DP), jnp.float32),
            pltpu.VMEM((EC, DP), jnp.float32),
            pltpu.VMEM((OB * DP,), jnp.float32),
            pltpu.SemaphoreType.DMA((2,)),
            pltpu.SemaphoreType.DMA((2,)),
            pltpu.SemaphoreType.DMA,
        ],
    )
    return f(a1s, cnts, adj2s, r1s, parms, feats_pad, rel_pad, w23)


def kernel(x, edge_index_all, rel_emb, r_index, line_graph_index,
           line_graph_val, Wx, Wr, Ww1):
    feats_pad = _matmul_pad(x, Wx, True, 2000)
    rel_pad = _matmul_pad(rel_emb, Wr, False, 2000)
    feats = feats_pad[:, :D]

    key = edge_index_all[0].astype(jnp.uint32) * np.uint32(N_NODES) \
        + edge_index_all[1].astype(jnp.uint32)
    sk = jnp.sort(key)
    first = jnp.concatenate([jnp.array([True]), sk[1:] != sk[:-1]])
    pos = jnp.cumsum(first) - 1
    n_u = (pos[-1] + 1).astype(jnp.int32)
    a1sort = (sk // N_NODES).astype(jnp.int32)
    a2sort = (sk % N_NODES).astype(jnp.int32)
    zi = jnp.zeros((NE,), jnp.int32)
    adj1s = zi.at[pos].add(jnp.where(first, a1sort, 0))
    adj2s = zi.at[pos].add(jnp.where(first, a2sort, 0))
    iota = jnp.arange(NE)
    valid = iota < n_u
    adj1pad = jnp.where(valid, adj1s, N_NODES)

    r0, r1 = r_index[0], r_index[1]
    cnts = jnp.zeros((NE,), jnp.int32).at[r0].add(1)
    rkey = jnp.sort(r0.astype(jnp.int32) * 2048 + (r1 % 2000).astype(jnp.int32))
    r0s = rkey >> 11
    r1s = rkey & 2047

    tt = jnp.arange(NW + 1)
    tgt = tt * n_u // NW
    cand = jnp.where(tgt >= n_u, N_NODES,
                     adj1pad[jnp.clip(tgt, 0, NE - 1)])
    nsplit = jnp.where(tt == 0, 0, (cand // 8) * 8).astype(jnp.int32)
    esplit = jnp.searchsorted(adj1pad, nsplit, side="left").astype(jnp.int32)
    rstart = jnp.searchsorted(r0s, esplit, side="left").astype(jnp.int32)
    parms = jnp.stack([esplit[:NW], esplit[1:], rstart[:NW],
                       nsplit[:NW], nsplit[1:]]).astype(jnp.int32)
    parms = jnp.pad(parms, ((0, 0), (0, 16)))

    zpad = jnp.zeros((PAD,), jnp.int32)
    a1sp = jnp.concatenate([adj1pad, zpad])
    cntsp = jnp.concatenate([cnts, zpad])
    adj2sp = jnp.concatenate([adj2s, zpad])
    r1sp = jnp.concatenate([r1s, zpad])
    w23 = jnp.concatenate([
        jnp.pad(Ww1[0, D:2 * D], (0, DP - D)),
        jnp.pad(Ww1[0, 2 * D:3 * D], (0, DP - D))])

    new_flat = _edge_pass(a1sp, cntsp, adj2sp, r1sp, parms, feats_pad,
                          rel_pad, w23)
    new_pad = new_flat.reshape(N_NODES, DP)
    return jnp.concatenate([feats, new_pad[:, :D]], axis=1)


# speedup vs baseline: 7.7437x; 1.2856x over previous
import functools

import jax
import jax.numpy as jnp
import numpy as np
from jax import lax
from jax.experimental import pallas as pl
from jax.experimental.pallas import tpu as pltpu
from jax.experimental.pallas import tpu_sc as plsc

N_NODES = 50000
NE = 800000
D = 100
DP = 128
NV = DP // 16
C = 128
EC = 128
OB = 64
NW = 32
PAD = 256


def _mm_kernel(act, x_ref, w_ref, o_ref):
    r = jnp.dot(x_ref[...], w_ref[...].T, preferred_element_type=jnp.float32)
    if act:
        r = jax.nn.leaky_relu(r, negative_slope=0.01)
    o_ref[:, :D] = r
    o_ref[:, D:] = jnp.zeros_like(o_ref[:, D:])


def _matmul_pad(x, W, act, bm):
    m, k = x.shape
    return pl.pallas_call(
        functools.partial(_mm_kernel, act),
        grid=(m // bm,),
        in_specs=[
            pl.BlockSpec((bm, k), lambda i: (i, 0)),
            pl.BlockSpec((D, k), lambda i: (0, 0)),
        ],
        out_specs=pl.BlockSpec((bm, DP), lambda i: (i, 0)),
        out_shape=jax.ShapeDtypeStruct((m, DP), jnp.float32),
    )(x, W)


def _rsqrt16(x):
    i = lax.bitcast_convert_type(x, jnp.int32)
    y = lax.bitcast_convert_type(jnp.int32(0x5F3759DF) - (i >> 1), jnp.float32)
    for _ in range(3):
        y = y * (1.5 - 0.5 * x * y * y)
    return y


def _splat(s):
    return jnp.full((16,), s, jnp.float32)


def _lsum(v):
    ii = lax.iota(jnp.int32, 16)
    for k in (1, 2, 4, 8):
        v = v + v.at[ii ^ k].get(mode="promise_in_bounds")
    return v


def _edge_body(a1s_hbm, cnts_hbm, adj2_hbm, r1s_hbm, parms_hbm, feats_hbm,
               rel_hbm, w23_hbm, out_hbm,
               parms_v, w23_v, a1_v, cnt_v, a2_v, r1_v, frows_v, rrows_v,
               obuf_v, sem, sem2, sem3):
    cid = lax.axis_index("c")
    sid = lax.axis_index("s")
    wid = sid * 2 + cid

    pltpu.sync_copy(parms_hbm, parms_v)
    pltpu.sync_copy(w23_hbm, w23_v)

    def _sca(j):
        return parms_v[j, pl.ds(wid, 16)][0]

    e_lo = _sca(0)
    e_hi = _sca(1)
    p0 = _sca(2)
    n_lo = _sca(3)
    n_hi = _sca(4)

    w2 = [w23_v[pl.ds(v * 16, 16)] for v in range(NV)]
    w3 = [w23_v[pl.ds(DP + v * 16, 16)] for v in range(NV)]

    zero = _splat(0.0)
    MNEG = -1e30
    mneg = _splat(MNEG)
    o7 = tuple(zero for _ in range(NV))
    CB = C + 16

    def finalize_to(cur, stop, st):
        @pl.loop(cur, stop, init_carry=st)
        def fin(nd, stf):
            obi, obase, m, s, O = stf
            inv = 1.0 / (s + 1e-16)
            base = obi * DP
            for v in range(NV):
                r = O[v] * inv
                obuf_v[pl.ds(base + v * 16, 16)] = jnp.where(
                    r >= 0.0, r, 0.01 * r)
            obi = obi + 1
            full = (obi == OB).astype(jnp.int32)

            @pl.loop(0, full, init_carry=0)
            def _fl(_, d):
                pltpu.async_copy(
                    obuf_v,
                    out_hbm.at[pl.ds(pl.multiple_of(obase * DP, 8),
                                     OB * DP)],
                    sem3).wait()
                return d

            obi = obi * (1 - full)
            obase = obase + OB * full
            return (obi, obase, mneg, zero, o7)

        return fin

    def issue(jx):
        par = jx & 1
        ebx = pl.multiple_of(jx * C, C)
        pltpu.async_copy(a1s_hbm.at[pl.ds(ebx, CB)],
                         a1_v.at[pl.ds(par * CB, CB)], sem.at[par])
        pltpu.async_copy(cnts_hbm.at[pl.ds(ebx, CB)],
                         cnt_v.at[pl.ds(par * CB, CB)], sem.at[par])
        pltpu.async_copy(adj2_hbm.at[pl.ds(ebx, C)],
                         a2_v.at[pl.ds(par * C, C)], sem2.at[par]).wait()
        pltpu.async_copy(feats_hbm.at[a2_v.at[pl.ds(par * C, C)]],
                         frows_v.at[pl.ds(par * C, C), :], sem2.at[par])

    def wait_chunk(jx):
        par = jx & 1
        ebx = pl.multiple_of(jx * C, C)
        pltpu.make_async_copy(a1s_hbm.at[pl.ds(ebx, CB)],
                              a1_v.at[pl.ds(par * CB, CB)],
                              sem.at[par]).wait()
        pltpu.make_async_copy(cnts_hbm.at[pl.ds(ebx, CB)],
                              cnt_v.at[pl.ds(par * CB, CB)],
                              sem.at[par]).wait()
        pltpu.make_async_copy(feats_hbm.at[a2_v.at[pl.ds(par * C, C)]],
                              frows_v.at[pl.ds(par * C, C), :],
                              sem2.at[par]).wait()

    jlo = e_lo >> 7
    jhi = (e_hi + (C - 1)) >> 7

    @pl.loop(jlo, jnp.minimum(jlo + 1, jhi), init_carry=0)
    def _prol(jx, d):
        issue(jx)
        return d

    st0 = (p0, p0 - EC, n_lo, 0, n_lo, mneg, zero, o7)

    @pl.loop(jlo, jhi, init_carry=st0)
    def chunk(j, st):
        p, pb, cur, obi, obase, m, s, O = st
        par = j & 1
        eb = pl.multiple_of(j * C, C)

        @pl.loop(j + 1, jnp.minimum(j + 2, jhi), init_carry=0)
        def _pref(jx, d):
            issue(jx)
            return d

        wait_chunk(j)
        abase = par * CB + 0
        fbase = par * C
        elo_j = jnp.maximum(e_lo, eb)
        ehi_j = jnp.minimum(e_hi, eb + C)

        @pl.loop(elo_j, ehi_j, init_carry=(p, pb, cur, obi, obase, m, s, O))
        def edges(e, st2):
            p, pb, cur, obi, obase, m, s, O = st2
            i = e - eb
            a1 = a1_v[pl.ds(abase + i, 16)][0]
            cnt = cnt_v[pl.ds(abase + i, 16)][0]

            obi, obase, m, s, O = finalize_to(
                cur, a1, (obi, obase, m, s, O))
            cur = a1

            @pl.loop(0, cnt, init_carry=(p, pb, o7))
            def ent(k, st3):
                p, pb, acc = st3
                need = (p - pb >= EC).astype(jnp.int32)

                @pl.loop(0, need, init_carry=pb)
                def refill(_, pbx):
                    npb = pl.multiple_of(p & ~7, 8)
                    pltpu.async_copy(r1s_hbm.at[pl.ds(npb, EC)], r1_v,
                                     sem3).wait()
                    pltpu.async_copy(rel_hbm.at[r1_v], rrows_v,
                                     sem3).wait()
                    return npb

                pb = refill
                bp = p - pb
                acc = tuple(acc[v] + rrows_v[bp, pl.ds(v * 16, 16)]
                            for v in range(NV))
                return p + 1, pb, acc

            p, pb, acc = ent

            n2 = zero
            for v in range(NV):
                n2 = n2 + acc[v] * acc[v]
            n2 = _lsum(n2)
            inv = jnp.where(n2 <= 1e-30, 0.0, _rsqrt16(n2))

            row = [frows_v[fbase + i, pl.ds(v * 16, 16)]
                   for v in range(NV)]
            t1 = zero
            t2 = zero
            t3 = zero
            t4 = zero
            for v in range(NV):
                t1 = t1 + row[v] * acc[v]
                t2 = t2 + row[v] * w2[v]
                t3 = t3 + acc[v] * w2[v]
                t4 = t4 + acc[v] * w3[v]
            ra = _lsum(t1)
            d2 = _lsum(t2)
            aw2 = _lsum(t3)
            aw3 = _lsum(t4)
            l = d2 - 2.0 * ra * aw2 * inv * inv + aw3 * inv

            mn = jnp.maximum(m, l)
            sc = jnp.exp(m - mn)
            w = jnp.exp(l - mn)
            s = s * sc + w
            ca = w * 2.0 * ra * inv * inv
            O = tuple(O[v] * sc + w * row[v] - ca * acc[v]
                      for v in range(NV))
            return p, pb, cur, obi, obase, mn, s, O

        return edges

    p, pb, cur, obi, obase, m, s, O = chunk
    obi, obase, m, s, O = finalize_to(cur, n_hi, (obi, obase, m, s, O))

    @pl.loop(0, obi, init_carry=0, step=8)
    def drain(jr, d):
        pltpu.async_copy(
            obuf_v.at[pl.ds(pl.multiple_of(jr * DP, 8), 8 * DP)],
            out_hbm.at[pl.ds(pl.multiple_of((obase + jr) * DP, 8), 8 * DP)],
            sem3).wait()
        return d


def _edge_pass(a1s, cnts, adj2s, r1s, parms, feats_pad, rel_pad, w23):
    mesh = plsc.VectorSubcoreMesh(core_axis_name="c", subcore_axis_name="s")
    f = pl.kernel(
        _edge_body,
        mesh=mesh,
        out_type=jax.ShapeDtypeStruct((N_NODES * DP,), jnp.float32),
        scratch_types=[
            pltpu.VMEM((5, NW + 16), jnp.int32),
            pltpu.VMEM((2 * DP,), jnp.float32),
            pltpu.VMEM((2 * (C + 16),), jnp.int32),
            pltpu.VMEM((2 * (C + 16),), jnp.int32),
            pltpu.VMEM((2 * C,), jnp.int32),
            pltpu.VMEM((EC,), jnp.int32),
            pltpu.VMEM((2 * C, DP), jnp.float32),
            pltpu.VMEM((EC, DP), jnp.float32),
            pltpu.VMEM((OB * DP,), jnp.float32),
            pltpu.SemaphoreType.DMA((2,)),
            pltpu.SemaphoreType.DMA((2,)),
            pltpu.SemaphoreType.DMA,
        ],
    )
    return f(a1s, cnts, adj2s, r1s, parms, feats_pad, rel_pad, w23)


def kernel(x, edge_index_all, rel_emb, r_index, line_graph_index,
           line_graph_val, Wx, Wr, Ww1):
    feats_pad = _matmul_pad(x, Wx, True, 2000)
    rel_pad = _matmul_pad(rel_emb, Wr, False, 2000)
    feats = feats_pad[:, :D]

    key = edge_index_all[0].astype(jnp.uint32) * np.uint32(N_NODES) \
        + edge_index_all[1].astype(jnp.uint32)
    sk = lax.sort([key], is_stable=False)[0]
    first = jnp.concatenate([jnp.array([True]), sk[1:] != sk[:-1]])
    pos = jnp.cumsum(first) - 1
    n_u = (pos[-1] + 1).astype(jnp.int32)
    a1sort = (sk // N_NODES).astype(jnp.int32)
    a2sort = (sk % N_NODES).astype(jnp.int32)
    zi = jnp.zeros((NE,), jnp.int32)
    adj1s = zi.at[pos].add(jnp.where(first, a1sort, 0))
    adj2s = zi.at[pos].add(jnp.where(first, a2sort, 0))
    iota = jnp.arange(NE)
    valid = iota < n_u
    adj1pad = jnp.where(valid, adj1s, N_NODES)

    r0, r1 = r_index[0], r_index[1]
    cnts = jnp.zeros((NE,), jnp.int32).at[r0].add(1)
    rkey = lax.sort(
        [r0.astype(jnp.int32) * 2048 + (r1 % 2000).astype(jnp.int32)],
        is_stable=False)[0]
    r0s = rkey >> 11
    r1s = rkey & 2047

    tt = jnp.arange(NW + 1)
    tgt = tt * n_u // NW
    cand = jnp.where(tgt >= n_u, N_NODES,
                     adj1pad[jnp.clip(tgt, 0, NE - 1)])
    nsplit = jnp.where(tt == 0, 0, (cand // 8) * 8).astype(jnp.int32)
    esplit = jnp.searchsorted(adj1pad, nsplit, side="left").astype(jnp.int32)
    rstart = jnp.searchsorted(r0s, esplit, side="left").astype(jnp.int32)
    parms = jnp.stack([esplit[:NW], esplit[1:], rstart[:NW],
                       nsplit[:NW], nsplit[1:]]).astype(jnp.int32)
    parms = jnp.pad(parms, ((0, 0), (0, 16)))

    zpad = jnp.zeros((PAD,), jnp.int32)
    a1sp = jnp.concatenate([adj1pad, zpad])
    cntsp = jnp.concatenate([cnts, zpad])
    adj2sp = jnp.concatenate([adj2s, zpad])
    r1sp = jnp.concatenate([r1s, zpad])
    w23 = jnp.concatenate([
        jnp.pad(Ww1[0, D:2 * D], (0, DP - D)),
        jnp.pad(Ww1[0, 2 * D:3 * D], (0, DP - D))])

    new_flat = _edge_pass(a1sp, cntsp, adj2sp, r1sp, parms, feats_pad,
                          rel_pad, w23)
    new_pad = new_flat.reshape(N_NODES, DP)
    return jnp.concatenate([feats, new_pad[:, :D]], axis=1)


# speedup vs baseline: 7.9970x; 1.0327x over previous
import functools

import jax
import jax.numpy as jnp
import numpy as np
from jax import lax
from jax.experimental import pallas as pl
from jax.experimental.pallas import tpu as pltpu
from jax.experimental.pallas import tpu_sc as plsc

N_NODES = 50000
NE = 800000
D = 100
DP = 128
NV = DP // 16
C = 128
EC = 128
OB = 64
NW = 32
PAD = 256


def _mm_kernel(act, x_ref, w_ref, o_ref):
    r = jnp.dot(x_ref[...], w_ref[...].T, preferred_element_type=jnp.float32)
    if act:
        r = jax.nn.leaky_relu(r, negative_slope=0.01)
    o_ref[:, :D] = r
    o_ref[:, D:] = jnp.zeros_like(o_ref[:, D:])


def _matmul_pad(x, W, act, bm):
    m, k = x.shape
    return pl.pallas_call(
        functools.partial(_mm_kernel, act),
        grid=(m // bm,),
        in_specs=[
            pl.BlockSpec((bm, k), lambda i: (i, 0)),
            pl.BlockSpec((D, k), lambda i: (0, 0)),
        ],
        out_specs=pl.BlockSpec((bm, DP), lambda i: (i, 0)),
        out_shape=jax.ShapeDtypeStruct((m, DP), jnp.float32),
    )(x, W)


def _rsqrt16(x):
    i = lax.bitcast_convert_type(x, jnp.int32)
    y = lax.bitcast_convert_type(jnp.int32(0x5F3759DF) - (i >> 1), jnp.float32)
    for _ in range(3):
        y = y * (1.5 - 0.5 * x * y * y)
    return y


def _splat(s):
    return jnp.full((16,), s, jnp.float32)


def _lsum(v):
    ii = lax.iota(jnp.int32, 16)
    for k in (1, 2, 4, 8):
        v = v + v.at[ii ^ k].get(mode="promise_in_bounds")
    return v


def _edge_body(a1s_hbm, cnts_hbm, adj2_hbm, r1s_hbm, parms_hbm, feats_hbm,
               rel_hbm, w23_hbm, out_hbm,
               parms_v, w23_v, a1_v, cnt_v, a2_v, r1_v, frows_v, rrows_v,
               obuf_v, sem, sem2, sem3):
    cid = lax.axis_index("c")
    sid = lax.axis_index("s")
    wid = sid * 2 + cid

    pltpu.sync_copy(parms_hbm, parms_v)
    pltpu.sync_copy(w23_hbm, w23_v)

    def _sca(j):
        return parms_v[j, pl.ds(wid, 16)][0]

    e_lo = _sca(0)
    e_hi = _sca(1)
    p0 = _sca(2)
    n_lo = _sca(3)
    n_hi = _sca(4)

    w2 = [w23_v[pl.ds(v * 16, 16)] for v in range(NV)]
    w3 = [w23_v[pl.ds(DP + v * 16, 16)] for v in range(NV)]

    zero = _splat(0.0)
    MNEG = -1e30
    mneg = _splat(MNEG)
    o7 = tuple(zero for _ in range(NV))
    CB = C + 16

    def finalize_to(cur, stop, st):
        @pl.loop(cur, stop, init_carry=st)
        def fin(nd, stf):
            obi, obase, m, s, O = stf
            inv = 1.0 / (s + 1e-16)
            base = obi * DP
            for v in range(NV):
                r = O[v] * inv
                obuf_v[pl.ds(base + v * 16, 16)] = jnp.where(
                    r >= 0.0, r, 0.01 * r)
            obi = obi + 1
            full = (obi == OB).astype(jnp.int32)

            @pl.loop(0, full, init_carry=0)
            def _fl(_, d):
                pltpu.async_copy(
                    obuf_v,
                    out_hbm.at[pl.ds(pl.multiple_of(obase * DP, 8),
                                     OB * DP)],
                    sem3).wait()
                return d

            obi = obi * (1 - full)
            obase = obase + OB * full
            return (obi, obase, mneg, zero, o7)

        return fin

    def issue(jx):
        par = jx & 1
        ebx = pl.multiple_of(jx * C, C)
        pltpu.async_copy(a1s_hbm.at[pl.ds(ebx, CB)],
                         a1_v.at[pl.ds(par * CB, CB)], sem.at[par])
        pltpu.async_copy(cnts_hbm.at[pl.ds(ebx, CB)],
                         cnt_v.at[pl.ds(par * CB, CB)], sem.at[par])
        pltpu.async_copy(adj2_hbm.at[pl.ds(ebx, C)],
                         a2_v.at[pl.ds(par * C, C)], sem2.at[par]).wait()
        pltpu.async_copy(feats_hbm.at[a2_v.at[pl.ds(par * C, C)]],
                         frows_v.at[pl.ds(par * C, C), :], sem2.at[par])

    def wait_chunk(jx):
        par = jx & 1
        ebx = pl.multiple_of(jx * C, C)
        pltpu.make_async_copy(a1s_hbm.at[pl.ds(ebx, CB)],
                              a1_v.at[pl.ds(par * CB, CB)],
                              sem.at[par]).wait()
        pltpu.make_async_copy(cnts_hbm.at[pl.ds(ebx, CB)],
                              cnt_v.at[pl.ds(par * CB, CB)],
                              sem.at[par]).wait()
        pltpu.make_async_copy(feats_hbm.at[a2_v.at[pl.ds(par * C, C)]],
                              frows_v.at[pl.ds(par * C, C), :],
                              sem2.at[par]).wait()

    jlo = e_lo >> 7
    jhi = (e_hi + (C - 1)) >> 7

    @pl.loop(jlo, jnp.minimum(jlo + 1, jhi), init_carry=0)
    def _prol(jx, d):
        issue(jx)
        return d

    st0 = (p0, p0 - EC, n_lo, 0, n_lo, mneg, zero, o7)

    @pl.loop(jlo, jhi, init_carry=st0)
    def chunk(j, st):
        p, pb, cur, obi, obase, m, s, O = st
        par = j & 1
        eb = pl.multiple_of(j * C, C)

        @pl.loop(j + 1, jnp.minimum(j + 2, jhi), init_carry=0)
        def _pref(jx, d):
            issue(jx)
            return d

        wait_chunk(j)
        abase = par * CB + 0
        fbase = par * C
        elo_j = jnp.maximum(e_lo, eb)
        ehi_j = jnp.minimum(e_hi, eb + C)

        @pl.loop(0, C, init_carry=(p, pb, cur, obi, obase, m, s, O),
                 unroll=2)
        def edges(ii, st2):
            p, pb, cur, obi, obase, m, s, O = st2
            e = eb + ii
            act = ((e >= elo_j) & (e < ehi_j)).astype(jnp.int32)
            i = ii
            a1 = a1_v[pl.ds(abase + i, 16)][0]
            cnt = cnt_v[pl.ds(abase + i, 16)][0]
            a1 = cur + act * (a1 - cur)
            cnt = cnt * act

            obi, obase, m, s, O = finalize_to(
                cur, a1, (obi, obase, m, s, O))
            cur = a1

            @pl.loop(0, cnt, init_carry=(p, pb, o7))
            def ent(k, st3):
                p, pb, acc = st3
                need = (p - pb >= EC).astype(jnp.int32)

                @pl.loop(0, need, init_carry=pb)
                def refill(_, pbx):
                    npb = pl.multiple_of(p & ~7, 8)
                    pltpu.async_copy(r1s_hbm.at[pl.ds(npb, EC)], r1_v,
                                     sem3).wait()
                    pltpu.async_copy(rel_hbm.at[r1_v], rrows_v,
                                     sem3).wait()
                    return npb

                pb = refill
                bp = p - pb
                acc = tuple(acc[v] + rrows_v[bp, pl.ds(v * 16, 16)]
                            for v in range(NV))
                return p + 1, pb, acc

            p, pb, acc = ent

            n2 = zero
            for v in range(NV):
                n2 = n2 + acc[v] * acc[v]
            n2 = _lsum(n2)
            inv = jnp.where(n2 <= 1e-30, 0.0, _rsqrt16(n2))

            row = [frows_v[fbase + i, pl.ds(v * 16, 16)]
                   for v in range(NV)]
            t1 = zero
            t2 = zero
            t3 = zero
            t4 = zero
            for v in range(NV):
                t1 = t1 + row[v] * acc[v]
                t2 = t2 + row[v] * w2[v]
                t3 = t3 + acc[v] * w2[v]
                t4 = t4 + acc[v] * w3[v]
            ra = _lsum(t1)
            d2 = _lsum(t2)
            aw2 = _lsum(t3)
            aw3 = _lsum(t4)
            l = d2 - 2.0 * ra * aw2 * inv * inv + aw3 * inv
            actf = _splat(act.astype(jnp.float32))
            l = l * actf + (1.0 - actf) * MNEG

            mn = jnp.maximum(m, l)
            sc = jnp.exp(m - mn)
            w = jnp.exp(l - mn) * actf
            s = s * sc + w
            ca = w * 2.0 * ra * inv * inv
            O = tuple(O[v] * sc + w * row[v] - ca * acc[v]
                      for v in range(NV))
            return p, pb, cur, obi, obase, mn, s, O

        return edges

    p, pb, cur, obi, obase, m, s, O = chunk
    obi, obase, m, s, O = finalize_to(cur, n_hi, (obi, obase, m, s, O))

    @pl.loop(0, obi, init_carry=0, step=8)
    def drain(jr, d):
        pltpu.async_copy(
            obuf_v.at[pl.ds(pl.multiple_of(jr * DP, 8), 8 * DP)],
            out_hbm.at[pl.ds(pl.multiple_of((obase + jr) * DP, 8), 8 * DP)],
            sem3).wait()
        return d


def _edge_pass(a1s, cnts, adj2s, r1s, parms, feats_pad, rel_pad, w23):
    mesh = plsc.VectorSubcoreMesh(core_axis_name="c", subcore_axis_name="s")
    f = pl.kernel(
        _edge_body,
        mesh=mesh,
        out_type=jax.ShapeDtypeStruct((N_NODES * DP,), jnp.float32),
        scratch_types=[
            pltpu.VMEM((5, NW + 16), jnp.int32),
            pltpu.VMEM((2 * DP,), jnp.float32),
            pltpu.VMEM((2 * (C + 16),), jnp.int32),
            pltpu.VMEM((2 * (C + 16),), jnp.int32),
            pltpu.VMEM((2 * C,), jnp.int32),
            pltpu.VMEM((EC,), jnp.int32),
            pltpu.VMEM((2 * C, DP), jnp.float32),
            pltpu.VMEM((EC, DP), jnp.float32),
            pltpu.VMEM((OB * DP,), jnp.float32),
            pltpu.SemaphoreType.DMA((2,)),
            pltpu.SemaphoreType.DMA((2,)),
            pltpu.SemaphoreType.DMA,
        ],
    )
    return f(a1s, cnts, adj2s, r1s, parms, feats_pad, rel_pad, w23)


def kernel(x, edge_index_all, rel_emb, r_index, line_graph_index,
           line_graph_val, Wx, Wr, Ww1):
    feats_pad = _matmul_pad(x, Wx, True, 2000)
    rel_pad = _matmul_pad(rel_emb, Wr, False, 2000)
    feats = feats_pad[:, :D]

    key = edge_index_all[0].astype(jnp.uint32) * np.uint32(N_NODES) \
        + edge_index_all[1].astype(jnp.uint32)
    sk = lax.sort([key], is_stable=False)[0]
    first = jnp.concatenate([jnp.array([True]), sk[1:] != sk[:-1]])
    pos = jnp.cumsum(first) - 1
    n_u = (pos[-1] + 1).astype(jnp.int32)
    a1sort = (sk // N_NODES).astype(jnp.int32)
    a2sort = (sk % N_NODES).astype(jnp.int32)
    zi = jnp.zeros((NE,), jnp.int32)
    adj1s = zi.at[pos].add(jnp.where(first, a1sort, 0))
    adj2s = zi.at[pos].add(jnp.where(first, a2sort, 0))
    iota = jnp.arange(NE)
    valid = iota < n_u
    adj1pad = jnp.where(valid, adj1s, N_NODES)

    r0, r1 = r_index[0], r_index[1]
    cnts = jnp.zeros((NE,), jnp.int32).at[r0].add(1)
    rkey = lax.sort(
        [r0.astype(jnp.int32) * 2048 + (r1 % 2000).astype(jnp.int32)],
        is_stable=False)[0]
    r0s = rkey >> 11
    r1s = rkey & 2047

    tt = jnp.arange(NW + 1)
    tgt = tt * n_u // NW
    cand = jnp.where(tgt >= n_u, N_NODES,
                     adj1pad[jnp.clip(tgt, 0, NE - 1)])
    nsplit = jnp.where(tt == 0, 0, (cand // 8) * 8).astype(jnp.int32)
    esplit = jnp.searchsorted(adj1pad, nsplit, side="left").astype(jnp.int32)
    rstart = jnp.searchsorted(r0s, esplit, side="left").astype(jnp.int32)
    parms = jnp.stack([esplit[:NW], esplit[1:], rstart[:NW],
                       nsplit[:NW], nsplit[1:]]).astype(jnp.int32)
    parms = jnp.pad(parms, ((0, 0), (0, 16)))

    zpad = jnp.zeros((PAD,), jnp.int32)
    a1sp = jnp.concatenate([adj1pad, zpad])
    cntsp = jnp.concatenate([cnts, zpad])
    adj2sp = jnp.concatenate([adj2s, zpad])
    r1sp = jnp.concatenate([r1s, zpad])
    w23 = jnp.concatenate([
        jnp.pad(Ww1[0, D:2 * D], (0, DP - D)),
        jnp.pad(Ww1[0, 2 * D:3 * D], (0, DP - D))])

    new_flat = _edge_pass(a1sp, cntsp, adj2sp, r1sp, parms, feats_pad,
                          rel_pad, w23)
    new_pad = new_flat.reshape(N_NODES, DP)
    return jnp.concatenate([feats, new_pad[:, :D]], axis=1)


# speedup vs baseline: 8.2231x; 1.0283x over previous
import functools

import jax
import jax.numpy as jnp
import numpy as np
from jax import lax
from jax.experimental import pallas as pl
from jax.experimental.pallas import tpu as pltpu
from jax.experimental.pallas import tpu_sc as plsc

N_NODES = 50000
NE = 800000
D = 100
DP = 128
NV = DP // 16
C = 128
EC = 128
OB = 64
NW = 32
PAD = 256


def _mm_kernel(act, x_ref, w_ref, o_ref):
    r = jnp.dot(x_ref[...], w_ref[...].T, preferred_element_type=jnp.float32)
    if act:
        r = jax.nn.leaky_relu(r, negative_slope=0.01)
    o_ref[:, :D] = r
    o_ref[:, D:] = jnp.zeros_like(o_ref[:, D:])


def _matmul_pad(x, W, act, bm):
    m, k = x.shape
    return pl.pallas_call(
        functools.partial(_mm_kernel, act),
        grid=(m // bm,),
        in_specs=[
            pl.BlockSpec((bm, k), lambda i: (i, 0)),
            pl.BlockSpec((D, k), lambda i: (0, 0)),
        ],
        out_specs=pl.BlockSpec((bm, DP), lambda i: (i, 0)),
        out_shape=jax.ShapeDtypeStruct((m, DP), jnp.float32),
    )(x, W)


def _rsqrt16(x):
    i = lax.bitcast_convert_type(x, jnp.int32)
    y = lax.bitcast_convert_type(jnp.int32(0x5F3759DF) - (i >> 1), jnp.float32)
    for _ in range(3):
        y = y * (1.5 - 0.5 * x * y * y)
    return y


def _splat(s):
    return jnp.full((16,), s, jnp.float32)


def _lsum(v):
    ii = lax.iota(jnp.int32, 16)
    for k in (1, 2, 4, 8):
        v = v + v.at[ii ^ k].get(mode="promise_in_bounds")
    return v


def _edge_body(a1s_hbm, cnts_hbm, adj2_hbm, r1s_hbm, parms_hbm, feats_hbm,
               rel_hbm, w23_hbm, out_hbm,
               parms_v, w23_v, a1_v, cnt_v, a2_v, r1_v, frows_v, rrows_v,
               obuf_v, sem, sem2, sem3):
    cid = lax.axis_index("c")
    sid = lax.axis_index("s")
    wid = sid * 2 + cid

    pltpu.sync_copy(parms_hbm, parms_v)
    pltpu.sync_copy(w23_hbm, w23_v)

    def _sca(j):
        return parms_v[j, pl.ds(wid, 16)][0]

    e_lo = _sca(0)
    e_hi = _sca(1)
    p0 = _sca(2)
    n_lo = _sca(3)
    n_hi = _sca(4)

    w2 = [w23_v[pl.ds(v * 16, 16)] for v in range(NV)]
    w3 = [w23_v[pl.ds(DP + v * 16, 16)] for v in range(NV)]

    zero = _splat(0.0)
    MNEG = -1e30
    mneg = _splat(MNEG)
    o7 = tuple(zero for _ in range(NV))
    CB = C + 16

    def finalize_to(cur, stop, st):
        @pl.loop(cur, stop, init_carry=st)
        def fin(nd, stf):
            obi, obase, m, s, O = stf
            inv = 1.0 / (s + 1e-16)
            base = obi * DP
            for v in range(NV):
                r = O[v] * inv
                obuf_v[pl.ds(base + v * 16, 16)] = jnp.where(
                    r >= 0.0, r, 0.01 * r)
            obi = obi + 1
            full = (obi == OB).astype(jnp.int32)

            @pl.loop(0, full, init_carry=0)
            def _fl(_, d):
                pltpu.async_copy(
                    obuf_v,
                    out_hbm.at[pl.ds(pl.multiple_of(obase * DP, 8),
                                     OB * DP)],
                    sem3).wait()
                return d

            obi = obi * (1 - full)
            obase = obase + OB * full
            return (obi, obase, mneg, zero, o7)

        return fin

    def issue(jx):
        par = jx & 1
        ebx = pl.multiple_of(jx * C, C)
        pltpu.async_copy(a1s_hbm.at[pl.ds(ebx, CB)],
                         a1_v.at[pl.ds(par * CB, CB)], sem.at[par])
        pltpu.async_copy(cnts_hbm.at[pl.ds(ebx, CB)],
                         cnt_v.at[pl.ds(par * CB, CB)], sem.at[par])
        pltpu.async_copy(adj2_hbm.at[pl.ds(ebx, C)],
                         a2_v.at[pl.ds(par * C, C)], sem2.at[par]).wait()
        pltpu.async_copy(feats_hbm.at[a2_v.at[pl.ds(par * C, C)]],
                         frows_v.at[pl.ds(par * C, C), :], sem2.at[par])

    def wait_chunk(jx):
        par = jx & 1
        ebx = pl.multiple_of(jx * C, C)
        pltpu.make_async_copy(a1s_hbm.at[pl.ds(ebx, CB)],
                              a1_v.at[pl.ds(par * CB, CB)],
                              sem.at[par]).wait()
        pltpu.make_async_copy(cnts_hbm.at[pl.ds(ebx, CB)],
                              cnt_v.at[pl.ds(par * CB, CB)],
                              sem.at[par]).wait()
        pltpu.make_async_copy(feats_hbm.at[a2_v.at[pl.ds(par * C, C)]],
                              frows_v.at[pl.ds(par * C, C), :],
                              sem2.at[par]).wait()

    jlo = e_lo >> 7
    jhi = (e_hi + (C - 1)) >> 7

    @pl.loop(jlo, jnp.minimum(jlo + 1, jhi), init_carry=0)
    def _prol(jx, d):
        issue(jx)
        return d

    st0 = (p0, p0 - EC, n_lo, 0, n_lo, mneg, zero, o7)

    @pl.loop(jlo, jhi, init_carry=st0)
    def chunk(j, st):
        p, pb, cur, obi, obase, m, s, O = st
        par = j & 1
        eb = pl.multiple_of(j * C, C)

        @pl.loop(j + 1, jnp.minimum(j + 2, jhi), init_carry=0)
        def _pref(jx, d):
            issue(jx)
            return d

        wait_chunk(j)
        abase = par * CB + 0
        fbase = par * C
        elo_j = jnp.maximum(e_lo, eb)
        ehi_j = jnp.minimum(e_hi, eb + C)

        @pl.loop(0, C, init_carry=(p, pb, cur, obi, obase, m, s, O),
                 unroll=4)
        def edges(ii, st2):
            p, pb, cur, obi, obase, m, s, O = st2
            e = eb + ii
            act = ((e >= elo_j) & (e < ehi_j)).astype(jnp.int32)
            i = ii
            a1 = a1_v[pl.ds(abase + i, 16)][0]
            cnt = cnt_v[pl.ds(abase + i, 16)][0]
            a1 = cur + act * (a1 - cur)
            cnt = cnt * act

            obi, obase, m, s, O = finalize_to(
                cur, a1, (obi, obase, m, s, O))
            cur = a1

            @pl.loop(0, cnt, init_carry=(p, pb, o7))
            def ent(k, st3):
                p, pb, acc = st3
                need = (p - pb >= EC).astype(jnp.int32)

                @pl.loop(0, need, init_carry=pb)
                def refill(_, pbx):
                    npb = pl.multiple_of(p & ~7, 8)
                    pltpu.async_copy(r1s_hbm.at[pl.ds(npb, EC)], r1_v,
                                     sem3).wait()
                    pltpu.async_copy(rel_hbm.at[r1_v], rrows_v,
                                     sem3).wait()
                    return npb

                pb = refill
                bp = p - pb
                acc = tuple(acc[v] + rrows_v[bp, pl.ds(v * 16, 16)]
                            for v in range(NV))
                return p + 1, pb, acc

            p, pb, acc = ent

            n2 = zero
            for v in range(NV):
                n2 = n2 + acc[v] * acc[v]
            n2 = _lsum(n2)
            inv = jnp.where(n2 <= 1e-30, 0.0, _rsqrt16(n2))

            row = [frows_v[fbase + i, pl.ds(v * 16, 16)]
                   for v in range(NV)]
            t1 = zero
            t2 = zero
            t3 = zero
            t4 = zero
            for v in range(NV):
                t1 = t1 + row[v] * acc[v]
                t2 = t2 + row[v] * w2[v]
                t3 = t3 + acc[v] * w2[v]
                t4 = t4 + acc[v] * w3[v]
            ra = _lsum(t1)
            d2 = _lsum(t2)
            aw2 = _lsum(t3)
            aw3 = _lsum(t4)
            l = d2 - 2.0 * ra * aw2 * inv * inv + aw3 * inv
            actf = _splat(act.astype(jnp.float32))
            l = l * actf + (1.0 - actf) * MNEG

            mn = jnp.maximum(m, l)
            sc = jnp.exp(m - mn)
            w = jnp.exp(l - mn) * actf
            s = s * sc + w
            ca = w * 2.0 * ra * inv * inv
            O = tuple(O[v] * sc + w * row[v] - ca * acc[v]
                      for v in range(NV))
            return p, pb, cur, obi, obase, mn, s, O

        return edges

    p, pb, cur, obi, obase, m, s, O = chunk
    obi, obase, m, s, O = finalize_to(cur, n_hi, (obi, obase, m, s, O))

    @pl.loop(0, obi, init_carry=0, step=8)
    def drain(jr, d):
        pltpu.async_copy(
            obuf_v.at[pl.ds(pl.multiple_of(jr * DP, 8), 8 * DP)],
            out_hbm.at[pl.ds(pl.multiple_of((obase + jr) * DP, 8), 8 * DP)],
            sem3).wait()
        return d


def _edge_pass(a1s, cnts, adj2s, r1s, parms, feats_pad, rel_pad, w23):
    mesh = plsc.VectorSubcoreMesh(core_axis_name="c", subcore_axis_name="s")
    f = pl.kernel(
        _edge_body,
        mesh=mesh,
        out_type=jax.ShapeDtypeStruct((N_NODES * DP,), jnp.float32),
        scratch_types=[
            pltpu.VMEM((5, NW + 16), jnp.int32),
            pltpu.VMEM((2 * DP,), jnp.float32),
            pltpu.VMEM((2 * (C + 16),), jnp.int32),
            pltpu.VMEM((2 * (C + 16),), jnp.int32),
            pltpu.VMEM((2 * C,), jnp.int32),
            pltpu.VMEM((EC,), jnp.int32),
            pltpu.VMEM((2 * C, DP), jnp.float32),
            pltpu.VMEM((EC, DP), jnp.float32),
            pltpu.VMEM((OB * DP,), jnp.float32),
            pltpu.SemaphoreType.DMA((2,)),
            pltpu.SemaphoreType.DMA((2,)),
            pltpu.SemaphoreType.DMA,
        ],
    )
    return f(a1s, cnts, adj2s, r1s, parms, feats_pad, rel_pad, w23)


def kernel(x, edge_index_all, rel_emb, r_index, line_graph_index,
           line_graph_val, Wx, Wr, Ww1):
    feats_pad = _matmul_pad(x, Wx, True, 2000)
    rel_pad = _matmul_pad(rel_emb, Wr, False, 2000)
    feats = feats_pad[:, :D]

    key = edge_index_all[0].astype(jnp.uint32) * np.uint32(N_NODES) \
        + edge_index_all[1].astype(jnp.uint32)
    sk = lax.sort([key], is_stable=False)[0]
    first = jnp.concatenate([jnp.array([True]), sk[1:] != sk[:-1]])
    pos = jnp.cumsum(first) - 1
    n_u = (pos[-1] + 1).astype(jnp.int32)
    a1sort = (sk // N_NODES).astype(jnp.int32)
    a2sort = (sk % N_NODES).astype(jnp.int32)
    zi = jnp.zeros((NE,), jnp.int32)
    adj1s = zi.at[pos].add(jnp.where(first, a1sort, 0))
    adj2s = zi.at[pos].add(jnp.where(first, a2sort, 0))
    iota = jnp.arange(NE)
    valid = iota < n_u
    adj1pad = jnp.where(valid, adj1s, N_NODES)

    r0, r1 = r_index[0], r_index[1]
    cnts = jnp.zeros((NE,), jnp.int32).at[r0].add(1)
    rkey = lax.sort(
        [r0.astype(jnp.int32) * 2048 + (r1 % 2000).astype(jnp.int32)],
        is_stable=False)[0]
    r0s = rkey >> 11
    r1s = rkey & 2047

    tt = jnp.arange(NW + 1)
    tgt = tt * n_u // NW
    cand = jnp.where(tgt >= n_u, N_NODES,
                     adj1pad[jnp.clip(tgt, 0, NE - 1)])
    nsplit = jnp.where(tt == 0, 0, (cand // 8) * 8).astype(jnp.int32)
    esplit = jnp.searchsorted(adj1pad, nsplit, side="left").astype(jnp.int32)
    rstart = jnp.searchsorted(r0s, esplit, side="left").astype(jnp.int32)
    parms = jnp.stack([esplit[:NW], esplit[1:], rstart[:NW],
                       nsplit[:NW], nsplit[1:]]).astype(jnp.int32)
    parms = jnp.pad(parms, ((0, 0), (0, 16)))

    zpad = jnp.zeros((PAD,), jnp.int32)
    a1sp = jnp.concatenate([adj1pad, zpad])
    cntsp = jnp.concatenate([cnts, zpad])
    adj2sp = jnp.concatenate([adj2s, zpad])
    r1sp = jnp.concatenate([r1s, zpad])
    w23 = jnp.concatenate([
        jnp.pad(Ww1[0, D:2 * D], (0, DP - D)),
        jnp.pad(Ww1[0, 2 * D:3 * D], (0, DP - D))])

    new_flat = _edge_pass(a1sp, cntsp, adj2sp, r1sp, parms, feats_pad,
                          rel_pad, w23)
    new_pad = new_flat.reshape(N_NODES, DP)
    return jnp.concatenate([feats, new_pad[:, :D]], axis=1)
